# Initial kernel scaffold; baseline (speedup 1.0000x reference)
import jax
import jax.numpy as jnp
from jax import lax
from jax.experimental import pallas as pl
from jax.experimental.pallas import tpu as pltpu
from jax.experimental.pallas import tpu_sc as plsc

N = 10000
E = 320000
CH = 128
NC = 2
NS = 16
NW = NC * NS
EPW = E // NW
CHK = 80
NCHK = EPW // CHK
NPAD = 10240
RPT = NPAD // NS


def _rsqrt_newton(d):
    di = plsc.bitcast(d, jnp.int32)
    yi = jnp.int32(0x5F3759DF) - lax.shift_right_arithmetic(di, 1)
    y = plsc.bitcast(yi, jnp.float32)
    for _ in range(4):
        y = y * (1.5 - 0.5 * d * y * y)
    return jnp.where(d > 0.0, y, 0.0)


def _prop_chunk_body(j, row_s, col_s, w_s, x_hbm, gbuf, acc, sem):
    pltpu.async_copy(x_hbm.at[row_s.at[j]], gbuf, sem).wait()
    jv = jnp.full((16,), j, jnp.int32)
    for e in range(CHK):
        ev = jnp.full((16,), e, jnp.int32)
        wspl = plsc.load_gather(w_s, [jv, ev])
        for g in range(CH // 16):
            sl = pl.ds(g * 16, 16)
            gbuf[e, sl] = gbuf[e, sl] * wspl
    pltpu.sync_copy(gbuf, acc.at[col_s.at[j]], add=True)


def _sc_kernel_a(row3, col3, ew3, h_hbm, z16, z128,
                 w3_out, txp_out,
                 acc_deg, dis_sh, acc,
                 row_s, col_s, ew_s, w_s, dis_t, dbuf, dvec, ebuf, gbuf, sem):
    c = lax.axis_index("c")
    s = lax.axis_index("s")
    iota16 = lax.iota(jnp.int32, 16)
    zero16i = jnp.zeros((16,), jnp.int32)

    pltpu.sync_copy(z16, acc_deg.at[pl.ds(s * RPT, RPT)])
    pltpu.sync_copy(z128, acc.at[pl.ds(s * RPT, RPT)])
    pltpu.sync_copy(z16.at[pl.ds(0, CHK)], ebuf)
    plsc.subcore_barrier()

    for slab_off in (0, NS):
        slab = s + slab_off
        pltpu.sync_copy(row3.at[slab], row_s)
        pltpu.sync_copy(ew3.at[slab], ew_s)

        def deg_chunk(j, carry):
            for g in range(CHK // 16):
                ewv = ew_s[j, pl.ds(g * 16, 16)]
                plsc.store_scatter(ebuf, [g * 16 + iota16, zero16i], ewv)
            pltpu.sync_copy(ebuf, acc_deg.at[row_s.at[j]], add=True)
            return carry

        lax.fori_loop(0, NCHK, deg_chunk, 0)
    plsc.subcore_barrier()

    pltpu.sync_copy(acc_deg.at[pl.ds(s * RPT, RPT)], dbuf)
    for t in range(RPT // 16):
        degv = plsc.load_gather(dbuf, [t * 16 + iota16, zero16i])
        dvec[pl.ds(t * 16, 16)] = _rsqrt_newton(degv)
    pltpu.sync_copy(dvec, dis_sh.at[pl.ds(s * RPT, RPT)])
    plsc.subcore_barrier()

    pltpu.sync_copy(dis_sh, dis_t)
    slab = c * NS + s
    pltpu.sync_copy(row3.at[slab], row_s)
    pltpu.sync_copy(col3.at[slab], col_s)
    pltpu.sync_copy(ew3.at[slab], ew_s)

    def prop_chunk(j, carry):
        for g in range(CHK // 16):
            sl = pl.ds(g * 16, 16)
            r = row_s[j, sl]
            cc = col_s[j, sl]
            ewv = ew_s[j, sl]
            dr = plsc.load_gather(dis_t, [r])
            dc = plsc.load_gather(dis_t, [cc])
            w_s[j, sl] = -(dr * ewv * dc)
        _prop_chunk_body(j, row_s, col_s, w_s, h_hbm, gbuf, acc, sem)
        return carry

    lax.fori_loop(0, NCHK, prop_chunk, 0)
    pltpu.sync_copy(w_s, w3_out.at[slab])
    plsc.subcore_barrier()

    pltpu.sync_copy(acc.at[pl.ds(s * RPT, RPT)], txp_out.at[c, pl.ds(s * RPT, RPT)])


def _sc_kernel_c(row3, col3, w3, x_hbm, z128,
                 txp_out,
                 acc,
                 row_s, col_s, w_s, gbuf, sem):
    c = lax.axis_index("c")
    s = lax.axis_index("s")
    pltpu.sync_copy(z128, acc.at[pl.ds(s * RPT, RPT)])
    plsc.subcore_barrier()

    slab = c * NS + s
    pltpu.sync_copy(row3.at[slab], row_s)
    pltpu.sync_copy(col3.at[slab], col_s)
    pltpu.sync_copy(w3.at[slab], w_s)

    def prop_chunk(j, carry):
        _prop_chunk_body(j, row_s, col_s, w_s, x_hbm, gbuf, acc, sem)
        return carry

    lax.fori_loop(0, NCHK, prop_chunk, 0)
    plsc.subcore_barrier()
    pltpu.sync_copy(acc.at[pl.ds(s * RPT, RPT)], txp_out.at[c, pl.ds(s * RPT, RPT)])


def _sc_prop_a(row3, col3, ew3, h):
    z16 = jnp.zeros((RPT, 16), jnp.float32)
    z128 = jnp.zeros((RPT, CH), jnp.float32)
    mesh = plsc.VectorSubcoreMesh(core_axis_name="c", subcore_axis_name="s")
    fn = pl.kernel(
        _sc_kernel_a,
        out_type=(
            jax.ShapeDtypeStruct((NW, NCHK, CHK), jnp.float32),
            jax.ShapeDtypeStruct((NC, NPAD, CH), jnp.float32),
        ),
        mesh=mesh,
        scratch_types=[
            pltpu.VMEM_SHARED((NPAD, 16), jnp.float32),
            pltpu.VMEM_SHARED((NPAD,), jnp.float32),
            pltpu.VMEM_SHARED((NPAD, CH), jnp.float32),
            pltpu.VMEM((NCHK, CHK), jnp.int32),
            pltpu.VMEM((NCHK, CHK), jnp.int32),
            pltpu.VMEM((NCHK, CHK), jnp.float32),
            pltpu.VMEM((NCHK, CHK), jnp.float32),
            pltpu.VMEM((NPAD,), jnp.float32),
            pltpu.VMEM((RPT, 16), jnp.float32),
            pltpu.VMEM((RPT,), jnp.float32),
            pltpu.VMEM((CHK, 16), jnp.float32),
            pltpu.VMEM((CHK, CH), jnp.float32),
            pltpu.SemaphoreType.DMA,
        ],
    )
    return fn(row3, col3, ew3, h, z16, z128)


def _sc_prop_c(row3, col3, w3, x):
    z128 = jnp.zeros((RPT, CH), jnp.float32)
    mesh = plsc.VectorSubcoreMesh(core_axis_name="c", subcore_axis_name="s")
    fn = pl.kernel(
        _sc_kernel_c,
        out_type=jax.ShapeDtypeStruct((NC, NPAD, CH), jnp.float32),
        mesh=mesh,
        scratch_types=[
            pltpu.VMEM_SHARED((NPAD, CH), jnp.float32),
            pltpu.VMEM((NCHK, CHK), jnp.int32),
            pltpu.VMEM((NCHK, CHK), jnp.int32),
            pltpu.VMEM((NCHK, CHK), jnp.float32),
            pltpu.VMEM((CHK, CH), jnp.float32),
            pltpu.SemaphoreType.DMA,
        ],
    )
    return fn(row3, col3, w3, x)


def _tc_combine_kernel(a_ref, b_ref, o_ref):
    o_ref[...] = a_ref[...] + b_ref[...]


def _tc_combine(p):
    a = p[0, :N]
    b = p[1, :N]
    bn = 1000
    return pl.pallas_call(
        _tc_combine_kernel,
        out_shape=jax.ShapeDtypeStruct((N, CH), jnp.float32),
        grid=(N // bn,),
        in_specs=[
            pl.BlockSpec((bn, CH), lambda i: (i, 0)),
            pl.BlockSpec((bn, CH), lambda i: (i, 0)),
        ],
        out_specs=pl.BlockSpec((bn, CH), lambda i: (i, 0)),
    )(a, b)


def _tc_gates_kernel(x_ref, h_ref, c_ref, t1_ref, p0_ref, p1_ref,
                     w_ref, bx_ref, bc_ref, hn_ref, cn_ref):
    x = x_ref[...]
    h = h_ref[...]
    t1 = t1_ref[...]
    t2 = 2.0 * (p0_ref[...] + p1_ref[...]) - h
    u = jnp.concatenate([x, h, t1, t2], axis=1)
    z = jnp.dot(u, w_ref[...], preferred_element_type=jnp.float32)
    z = z + bx_ref[...] + bc_ref[...]
    i_g = jax.nn.sigmoid(z[:, 0 * CH:1 * CH])
    f_g = jax.nn.sigmoid(z[:, 1 * CH:2 * CH])
    t_g = jnp.tanh(z[:, 2 * CH:3 * CH])
    o_g = jax.nn.sigmoid(z[:, 3 * CH:4 * CH])
    c_new = f_g * c_ref[...] + i_g * t_g
    hn_ref[...] = o_g * jnp.tanh(c_new)
    cn_ref[...] = c_new


def _tc_gates(x, h, c, t1, p2, w_all, bx, bc):
    p0 = p2[0, :N]
    p1 = p2[1, :N]
    bn = 1000
    return pl.pallas_call(
        _tc_gates_kernel,
        out_shape=(
            jax.ShapeDtypeStruct((N, CH), jnp.float32),
            jax.ShapeDtypeStruct((N, CH), jnp.float32),
        ),
        grid=(N // bn,),
        in_specs=[
            pl.BlockSpec((bn, CH), lambda i: (i, 0)),
            pl.BlockSpec((bn, CH), lambda i: (i, 0)),
            pl.BlockSpec((bn, CH), lambda i: (i, 0)),
            pl.BlockSpec((bn, CH), lambda i: (i, 0)),
            pl.BlockSpec((bn, CH), lambda i: (i, 0)),
            pl.BlockSpec((bn, CH), lambda i: (i, 0)),
            pl.BlockSpec((4 * CH, 4 * CH), lambda i: (0, 0)),
            pl.BlockSpec((1, 4 * CH), lambda i: (0, 0)),
            pl.BlockSpec((1, 4 * CH), lambda i: (0, 0)),
        ],
        out_specs=(
            pl.BlockSpec((bn, CH), lambda i: (i, 0)),
            pl.BlockSpec((bn, CH), lambda i: (i, 0)),
        ),
    )(x, h, c, t1, p0, p1, w_all, bx, bc)


@jax.jit
def kernel(X, edge_index, edge_weight, H, C,
           W_i, b_i, W_f, b_f, W_c, b_c, W_o, b_o,
           conv_i_W, conv_i_b, conv_f_W, conv_f_b,
           conv_c_W, conv_c_b, conv_o_W, conv_o_b):
    row3 = edge_index[0].reshape(NW, NCHK, CHK)
    col3 = edge_index[1].reshape(NW, NCHK, CHK)
    ew3 = edge_weight.reshape(NW, NCHK, CHK)

    w3, t1p = _sc_prop_a(row3, col3, ew3, H)
    t1 = _tc_combine(t1p)
    t2p = _sc_prop_c(row3, col3, w3, t1)

    w_all = jnp.concatenate([
        jnp.concatenate([W_i, W_f, W_c, W_o], axis=1),
        jnp.concatenate([conv_i_W[0], conv_f_W[0], conv_c_W[0], conv_o_W[0]], axis=1),
        jnp.concatenate([conv_i_W[1], conv_f_W[1], conv_c_W[1], conv_o_W[1]], axis=1),
        jnp.concatenate([conv_i_W[2], conv_f_W[2], conv_c_W[2], conv_o_W[2]], axis=1),
    ], axis=0)
    bx = jnp.concatenate([b_i, b_f, b_c, b_o], axis=1)
    bc = jnp.concatenate([conv_i_b, conv_f_b, conv_c_b, conv_o_b])[None, :]

    h_new, c_new = _tc_gates(X, H, C, t1, t2p, w_all, bx, bc)
    return (h_new, c_new)


# speedup vs baseline: 12.5940x; 12.5940x over previous
import jax
import jax.numpy as jnp
from jax import lax
from jax.experimental import pallas as pl
from jax.experimental.pallas import tpu as pltpu
from jax.experimental.pallas import tpu_sc as plsc

N = 10000
E = 320000
CH = 128
NC = 2
NS = 16
NW = NC * NS
EPW = E // NW
CHK = 80
NCHK = EPW // CHK
NPAD = 10240
RPT = NPAD // NS

_SPLAT_DNUMS = lax.GatherDimensionNumbers(
    offset_dims=(), collapsed_slice_dims=(0,), start_index_map=(0,))


def _splat_lane(v, i):
    return lax.gather(v, jnp.full((16, 1), i, jnp.int32), _SPLAT_DNUMS, (1,),
                      mode=lax.GatherScatterMode.PROMISE_IN_BOUNDS)


def _rsqrt_newton(d):
    di = plsc.bitcast(d, jnp.int32)
    yi = jnp.int32(0x5F3759DF) - lax.shift_right_arithmetic(di, 1)
    y = plsc.bitcast(yi, jnp.float32)
    for _ in range(4):
        y = y * (1.5 - 0.5 * d * y * y)
    return jnp.where(d > 0.0, y, 0.0)


def _prop_chunk_body(j, row_s, col_s, f_s, x_hbm, gbuf, acc, sem):
    pltpu.async_copy(x_hbm.at[row_s.at[j]], gbuf, sem).wait()
    for g in range(CHK // 16):
        fv = f_s[j, pl.ds(g * 16, 16)]
        for i in range(16):
            e = g * 16 + i
            fspl = _splat_lane(fv, i)
            for k in range(CH // 16):
                sl = pl.ds(k * 16, 16)
                gbuf[e, sl] = gbuf[e, sl] * fspl
    pltpu.sync_copy(gbuf, acc.at[col_s.at[j]], add=True)


def _sc_kernel_a1(row3, ew3, z16,
                  dis_out, f3_out,
                  acc_deg, dis_sh,
                  row_s, ew_s, f_s, dis_t, dbuf, dvec, ebuf, sem):
    c = lax.axis_index("c")
    s = lax.axis_index("s")
    iota16 = lax.iota(jnp.int32, 16)
    zero16i = jnp.zeros((16,), jnp.int32)

    pltpu.sync_copy(z16, acc_deg.at[pl.ds(s * RPT, RPT)])
    plsc.subcore_barrier()

    for slab_off in (0, NS):
        slab = s + slab_off
        pltpu.sync_copy(row3.at[slab], row_s)
        pltpu.sync_copy(ew3.at[slab], ew_s)

        def deg_chunk(j, carry):
            for g in range(CHK // 16):
                ewv = ew_s[j, pl.ds(g * 16, 16)]
                for i in range(16):
                    ebuf[g * 16 + i, :] = _splat_lane(ewv, i)
            pltpu.sync_copy(ebuf, acc_deg.at[row_s.at[j]], add=True)
            return carry

        lax.fori_loop(0, NCHK, deg_chunk, 0)
    plsc.subcore_barrier()

    pltpu.sync_copy(acc_deg.at[pl.ds(s * RPT, RPT)], dbuf)
    for t in range(RPT // 16):
        degv = plsc.load_gather(dbuf, [t * 16 + iota16, zero16i])
        dvec[pl.ds(t * 16, 16)] = _rsqrt_newton(degv)
    pltpu.sync_copy(dvec, dis_sh.at[pl.ds(s * RPT, RPT)])

    @pl.when(c == 0)
    def _():
        pltpu.sync_copy(dvec, dis_out.at[pl.ds(s * RPT, RPT)])

    plsc.subcore_barrier()

    pltpu.sync_copy(dis_sh, dis_t)
    slab = c * NS + s
    pltpu.sync_copy(row3.at[slab], row_s)
    pltpu.sync_copy(ew3.at[slab], ew_s)

    def f_chunk(j, carry):
        for g in range(CHK // 16):
            sl = pl.ds(g * 16, 16)
            dr = plsc.load_gather(dis_t, [row_s[j, sl]])
            f_s[j, sl] = ew_s[j, sl] * dr
        return carry

    lax.fori_loop(0, NCHK, f_chunk, 0)
    pltpu.sync_copy(f_s, f3_out.at[slab])


def _sc_kernel_c(row3, col3, f3, x_hbm, z128,
                 txp_out,
                 acc,
                 row_s, col_s, f_s, gbuf, sem):
    c = lax.axis_index("c")
    s = lax.axis_index("s")
    pltpu.sync_copy(z128, acc.at[pl.ds(s * RPT, RPT)])
    plsc.subcore_barrier()

    slab = c * NS + s
    pltpu.sync_copy(row3.at[slab], row_s)
    pltpu.sync_copy(col3.at[slab], col_s)
    pltpu.sync_copy(f3.at[slab], f_s)

    def prop_chunk(j, carry):
        _prop_chunk_body(j, row_s, col_s, f_s, x_hbm, gbuf, acc, sem)
        return carry

    lax.fori_loop(0, NCHK, prop_chunk, 0)
    plsc.subcore_barrier()
    pltpu.sync_copy(acc.at[pl.ds(s * RPT, RPT)], txp_out.at[c, pl.ds(s * RPT, RPT)])


def _sc_deg_f(row3, ew3):
    z16 = jnp.zeros((RPT, 16), jnp.float32)
    mesh = plsc.VectorSubcoreMesh(core_axis_name="c", subcore_axis_name="s")
    fn = pl.kernel(
        _sc_kernel_a1,
        out_type=(
            jax.ShapeDtypeStruct((NPAD,), jnp.float32),
            jax.ShapeDtypeStruct((NW, NCHK, CHK), jnp.float32),
        ),
        mesh=mesh,
        scratch_types=[
            pltpu.VMEM_SHARED((NPAD, 16), jnp.float32),
            pltpu.VMEM_SHARED((NPAD,), jnp.float32),
            pltpu.VMEM((NCHK, CHK), jnp.int32),
            pltpu.VMEM((NCHK, CHK), jnp.float32),
            pltpu.VMEM((NCHK, CHK), jnp.float32),
            pltpu.VMEM((NPAD,), jnp.float32),
            pltpu.VMEM((RPT, 16), jnp.float32),
            pltpu.VMEM((RPT,), jnp.float32),
            pltpu.VMEM((CHK, 16), jnp.float32),
            pltpu.SemaphoreType.DMA,
        ],
        compiler_params=pltpu.CompilerParams(needs_layout_passes=False, use_tc_tiling_on_sc=False),
    )
    return fn(row3, ew3, z16)


def _sc_prop_c(row3, col3, f3, x):
    z128 = jnp.zeros((RPT, CH), jnp.float32)
    mesh = plsc.VectorSubcoreMesh(core_axis_name="c", subcore_axis_name="s")
    fn = pl.kernel(
        _sc_kernel_c,
        out_type=jax.ShapeDtypeStruct((NC, NPAD, CH), jnp.float32),
        mesh=mesh,
        scratch_types=[
            pltpu.VMEM_SHARED((NPAD, CH), jnp.float32),
            pltpu.VMEM((NCHK, CHK), jnp.int32),
            pltpu.VMEM((NCHK, CHK), jnp.int32),
            pltpu.VMEM((NCHK, CHK), jnp.float32),
            pltpu.VMEM((CHK, CH), jnp.float32),
            pltpu.SemaphoreType.DMA,
        ],
        compiler_params=pltpu.CompilerParams(needs_layout_passes=False, use_tc_tiling_on_sc=False),
    )
    return fn(row3, col3, f3, x, z128)


def _tc_combine_kernel(a_ref, b_ref, d_ref, o_ref):
    o_ref[...] = -d_ref[...] * (a_ref[...] + b_ref[...])


def _tc_combine(p, dis2):
    a = p[0, :N]
    b = p[1, :N]
    bn = 1000
    return pl.pallas_call(
        _tc_combine_kernel,
        out_shape=jax.ShapeDtypeStruct((N, CH), jnp.float32),
        grid=(N // bn,),
        in_specs=[
            pl.BlockSpec((bn, CH), lambda i: (i, 0)),
            pl.BlockSpec((bn, CH), lambda i: (i, 0)),
            pl.BlockSpec((bn, 1), lambda i: (i, 0)),
        ],
        out_specs=pl.BlockSpec((bn, CH), lambda i: (i, 0)),
    )(a, b, dis2)


def _tc_gates_kernel(x_ref, h_ref, c_ref, t1_ref, q0_ref, q1_ref, d_ref,
                     w_ref, bx_ref, bc_ref, hn_ref, cn_ref):
    x = x_ref[...]
    h = h_ref[...]
    t1 = t1_ref[...]
    t2 = -2.0 * d_ref[...] * (q0_ref[...] + q1_ref[...]) - h
    u = jnp.concatenate([x, h, t1, t2], axis=1)
    z = jnp.dot(u, w_ref[...], preferred_element_type=jnp.float32)
    z = z + bx_ref[...] + bc_ref[...]
    i_g = jax.nn.sigmoid(z[:, 0 * CH:1 * CH])
    f_g = jax.nn.sigmoid(z[:, 1 * CH:2 * CH])
    t_g = jnp.tanh(z[:, 2 * CH:3 * CH])
    o_g = jax.nn.sigmoid(z[:, 3 * CH:4 * CH])
    c_new = f_g * c_ref[...] + i_g * t_g
    hn_ref[...] = o_g * jnp.tanh(c_new)
    cn_ref[...] = c_new


def _tc_gates(x, h, c, t1, q, dis2, w_all, bx, bc):
    q0 = q[0, :N]
    q1 = q[1, :N]
    bn = 1000
    return pl.pallas_call(
        _tc_gates_kernel,
        out_shape=(
            jax.ShapeDtypeStruct((N, CH), jnp.float32),
            jax.ShapeDtypeStruct((N, CH), jnp.float32),
        ),
        grid=(N // bn,),
        in_specs=[
            pl.BlockSpec((bn, CH), lambda i: (i, 0)),
            pl.BlockSpec((bn, CH), lambda i: (i, 0)),
            pl.BlockSpec((bn, CH), lambda i: (i, 0)),
            pl.BlockSpec((bn, CH), lambda i: (i, 0)),
            pl.BlockSpec((bn, CH), lambda i: (i, 0)),
            pl.BlockSpec((bn, CH), lambda i: (i, 0)),
            pl.BlockSpec((bn, 1), lambda i: (i, 0)),
            pl.BlockSpec((4 * CH, 4 * CH), lambda i: (0, 0)),
            pl.BlockSpec((1, 4 * CH), lambda i: (0, 0)),
            pl.BlockSpec((1, 4 * CH), lambda i: (0, 0)),
        ],
        out_specs=(
            pl.BlockSpec((bn, CH), lambda i: (i, 0)),
            pl.BlockSpec((bn, CH), lambda i: (i, 0)),
        ),
    )(x, h, c, t1, q0, q1, dis2, w_all, bx, bc)


@jax.jit
def kernel(X, edge_index, edge_weight, H, C,
           W_i, b_i, W_f, b_f, W_c, b_c, W_o, b_o,
           conv_i_W, conv_i_b, conv_f_W, conv_f_b,
           conv_c_W, conv_c_b, conv_o_W, conv_o_b):
    row3 = edge_index[0].reshape(NW, NCHK, CHK)
    col3 = edge_index[1].reshape(NW, NCHK, CHK)
    ew3 = edge_weight.reshape(NW, NCHK, CHK)

    dis, f3 = _sc_deg_f(row3, ew3)
    t1p = _sc_prop_c(row3, col3, f3, H)
    dis2 = dis[:N].reshape(N, 1)
    t1 = _tc_combine(t1p, dis2)
    t2p = _sc_prop_c(row3, col3, f3, t1)

    w_all = jnp.concatenate([
        jnp.concatenate([W_i, W_f, W_c, W_o], axis=1),
        jnp.concatenate([conv_i_W[0], conv_f_W[0], conv_c_W[0], conv_o_W[0]], axis=1),
        jnp.concatenate([conv_i_W[1], conv_f_W[1], conv_c_W[1], conv_o_W[1]], axis=1),
        jnp.concatenate([conv_i_W[2], conv_f_W[2], conv_c_W[2], conv_o_W[2]], axis=1),
    ], axis=0)
    bx = jnp.concatenate([b_i, b_f, b_c, b_o], axis=1)
    bc = jnp.concatenate([conv_i_b, conv_f_b, conv_c_b, conv_o_b])[None, :]

    h_new, c_new = _tc_gates(X, H, C, t1, t2p, dis2, w_all, bx, bc)
    return (h_new, c_new)
